# baseline (speedup 1.0000x reference)
import functools

import jax
import jax.numpy as jnp
import numpy as np
from jax.experimental import pallas as pl
from jax.experimental.pallas import tpu as pltpu

D_MODEL = 128
MAX_LEN = 512


def _position_indices(shapes):
    D = shapes[0][1] - 1
    idx_list = []
    for s in shapes:
        t = s[1] - 1
        indices = np.concatenate(
            [np.zeros([1]), np.linspace(1, D, t).astype(np.int32)]
        )
        idx_list.append(indices.astype(np.int64))
    return np.concatenate(idx_list, axis=0).astype(np.int32)


_R_BLK = 304


def _make_fused_gather_broadcast(
    total: int, batch: int, vocab: int, ident_steps: int
):
    n_steps = -(-total // _R_BLK)

    def body(idx_ref, emb_ref, out_ref):
        pid = pl.program_id(0)

        def emit(rows):
            out_ref[...] = jnp.broadcast_to(
                rows[:, None, :], (_R_BLK, batch, D_MODEL)
            )

        if ident_steps:
            @pl.when(pid < ident_steps)
            def _ident():
                base = pl.multiple_of(pid * _R_BLK, _R_BLK)
                emit(emb_ref[pl.ds(base, _R_BLK)])

        @pl.when(pid >= ident_steps)
        def _gather():
            cols = jax.lax.broadcasted_iota(jnp.int32, (_R_BLK, vocab), 1)
            onehot = (cols == idx_ref[...]).astype(jnp.float32)
            emit(jnp.dot(
                onehot, emb_ref[...],
                preferred_element_type=jnp.float32,
                precision=jax.lax.Precision.HIGHEST,
            ))

    return pl.pallas_call(
        body,
        grid=(n_steps,),
        in_specs=[
            pl.BlockSpec((_R_BLK, 1), lambda r: (r, 0)),
            pl.BlockSpec((vocab, D_MODEL), lambda r: (0, 0)),
        ],
        out_specs=pl.BlockSpec((_R_BLK, batch, D_MODEL), lambda r: (r, 0, 0)),
        out_shape=jax.ShapeDtypeStruct((total, batch, D_MODEL), jnp.float32),
    )


def _make_manual_pipeline(total: int, batch: int, vocab: int,
                          chunks, n_bufs: int = 5):
    c_max = max(c[1] for c in chunks)

    def body(idx_ref, emb_ref, out_ref, bufs, *sems):
        handles = [None] * n_bufs
        for i, (start, size, ident) in enumerate(chunks):
            k = i % n_bufs
            if handles[k] is not None:
                handles[k].wait()
            if ident:
                rows = emb_ref[pl.ds(start, size)]
            else:
                cols = jax.lax.broadcasted_iota(jnp.int32, (size, vocab), 1)
                onehot = (cols == idx_ref[pl.ds(start, size)]).astype(
                    jnp.float32)
                rows = jnp.dot(
                    onehot, emb_ref[...],
                    preferred_element_type=jnp.float32,
                    precision=jax.lax.Precision.HIGHEST,
                )
            bufs[k, pl.ds(0, size)] = jnp.broadcast_to(
                rows[:, None, :], (size, batch, D_MODEL))
            handles[k] = pltpu.make_async_copy(
                bufs.at[k, pl.ds(0, size)],
                out_ref.at[pl.ds(start, size)],
                sems[k],
            )
            handles[k].start()
        for h in handles:
            if h is not None:
                h.wait()

    return pl.pallas_call(
        body,
        in_specs=[
            pl.BlockSpec(memory_space=pltpu.VMEM),
            pl.BlockSpec(memory_space=pltpu.VMEM),
        ],
        out_specs=pl.BlockSpec(memory_space=pl.ANY),
        out_shape=jax.ShapeDtypeStruct((total, batch, D_MODEL), jnp.float32),
        scratch_shapes=(
            [pltpu.VMEM((n_bufs, c_max, batch, D_MODEL), jnp.float32)]
            + [pltpu.SemaphoreType.DMA] * n_bufs
        ),
    )


def kernel(modal_feat_0, modal_feat_1, modal_feat_2, emb):
    shapes = [modal_feat_0.shape, modal_feat_1.shape, modal_feat_2.shape]
    batch = shapes[0][0]
    idx = _position_indices(shapes)
    total = idx.shape[0]

    idx_col = np.zeros((total, 1), np.int32)
    idx_col[:, 0] = idx

    eq = idx == np.arange(total, dtype=np.int32)
    ident_prefix = total if eq.all() else int(np.argmin(eq))

    chunks = []
    plan = [32, 96, 128, 256, 200, 186]
    pos = 0
    for size in plan:
        size = min(size, total - pos)
        if size <= 0:
            break
        if pos + size > ident_prefix > pos:
            size = ident_prefix - pos
        chunks.append((pos, size, pos + size <= ident_prefix))
        pos += size
    while pos < total:
        size = min(256, total - pos)
        chunks.append((pos, size, pos + size <= ident_prefix))
        pos += size

    fused = _make_manual_pipeline(total, batch, emb.shape[0], chunks)
    out_t = fused(jnp.asarray(idx_col), emb)
    return jnp.transpose(out_t, (1, 0, 2))


# speedup vs baseline: 1.0943x; 1.0943x over previous
import functools

import jax
import jax.numpy as jnp
import numpy as np
from jax.experimental import pallas as pl
from jax.experimental.pallas import tpu as pltpu

D_MODEL = 128
MAX_LEN = 512


def _position_indices(shapes):
    D = shapes[0][1] - 1
    idx_list = []
    for s in shapes:
        t = s[1] - 1
        indices = np.concatenate(
            [np.zeros([1]), np.linspace(1, D, t).astype(np.int32)]
        )
        idx_list.append(indices.astype(np.int64))
    return np.concatenate(idx_list, axis=0).astype(np.int32)


_R_BLK = 304


def _make_fused_gather_broadcast(
    total: int, batch: int, vocab: int, ident_steps: int
):
    n_steps = -(-total // _R_BLK)

    def body(idx_ref, emb_ref, out_ref):
        pid = pl.program_id(0)

        def emit(rows):
            out_ref[...] = jnp.broadcast_to(
                rows[:, None, :], (_R_BLK, batch, D_MODEL)
            )

        if ident_steps:
            @pl.when(pid < ident_steps)
            def _ident():
                base = pl.multiple_of(pid * _R_BLK, _R_BLK)
                emit(emb_ref[pl.ds(base, _R_BLK)])

        @pl.when(pid >= ident_steps)
        def _gather():
            cols = jax.lax.broadcasted_iota(jnp.int32, (_R_BLK, vocab), 1)
            onehot = (cols == idx_ref[...]).astype(jnp.float32)
            emit(jnp.dot(
                onehot, emb_ref[...],
                preferred_element_type=jnp.float32,
                precision=jax.lax.Precision.HIGHEST,
            ))

    return pl.pallas_call(
        body,
        grid=(n_steps,),
        in_specs=[
            pl.BlockSpec((_R_BLK, 1), lambda r: (r, 0)),
            pl.BlockSpec((vocab, D_MODEL), lambda r: (0, 0)),
        ],
        out_specs=pl.BlockSpec((_R_BLK, batch, D_MODEL), lambda r: (r, 0, 0)),
        out_shape=jax.ShapeDtypeStruct((total, batch, D_MODEL), jnp.float32),
    )


def _make_manual_pipeline(total: int, batch: int, vocab: int,
                          chunks, n_bufs: int = 5):
    c_max = max(c[1] for c in chunks)

    def body(idx_ref, emb_ref, out_ref, bufs, *sems):
        handles = [None] * n_bufs
        for i, (start, size, ident) in enumerate(chunks):
            k = i % n_bufs
            if handles[k] is not None:
                handles[k].wait()
            if ident:
                rows = emb_ref[pl.ds(start, size)]
            else:
                cols = jax.lax.broadcasted_iota(jnp.int32, (size, vocab), 1)
                onehot = (cols == idx_ref[pl.ds(start, size)]).astype(
                    jnp.float32)
                rows = jnp.dot(
                    onehot, emb_ref[...],
                    preferred_element_type=jnp.float32,
                    precision=jax.lax.Precision.HIGHEST,
                )
            bufs[k, pl.ds(0, size)] = jnp.broadcast_to(
                rows[:, None, :], (size, batch, D_MODEL))
            handles[k] = pltpu.make_async_copy(
                bufs.at[k, pl.ds(0, size)],
                out_ref.at[pl.ds(start, size)],
                sems[k],
            )
            handles[k].start()
        for h in handles:
            if h is not None:
                h.wait()

    return pl.pallas_call(
        body,
        in_specs=[
            pl.BlockSpec(memory_space=pltpu.VMEM),
            pl.BlockSpec(memory_space=pltpu.VMEM),
        ],
        out_specs=pl.BlockSpec(memory_space=pl.ANY),
        out_shape=jax.ShapeDtypeStruct((total, batch, D_MODEL), jnp.float32),
        scratch_shapes=(
            [pltpu.VMEM((n_bufs, c_max, batch, D_MODEL), jnp.float32)]
            + [pltpu.SemaphoreType.DMA] * n_bufs
        ),
    )


def kernel(modal_feat_0, modal_feat_1, modal_feat_2, emb):
    shapes = [modal_feat_0.shape, modal_feat_1.shape, modal_feat_2.shape]
    batch = shapes[0][0]
    idx = _position_indices(shapes)
    total = idx.shape[0]

    idx_col = np.zeros((total, 1), np.int32)
    idx_col[:, 0] = idx

    eq = idx == np.arange(total, dtype=np.int32)
    ident_prefix = total if eq.all() else int(np.argmin(eq))

    chunks = []
    plan = [32, 96, 128, 128, 128, 136, 128, 122]
    pos = 0
    for size in plan:
        size = min(size, total - pos)
        if size <= 0:
            break
        if pos + size > ident_prefix > pos:
            size = ident_prefix - pos
        chunks.append((pos, size, pos + size <= ident_prefix))
        pos += size
    while pos < total:
        size = min(256, total - pos)
        chunks.append((pos, size, pos + size <= ident_prefix))
        pos += size

    fused = _make_manual_pipeline(total, batch, emb.shape[0], chunks)
    out_t = fused(jnp.asarray(idx_col), emb)
    return jnp.transpose(out_t, (1, 0, 2))


# speedup vs baseline: 1.1119x; 1.0161x over previous
import functools

import jax
import jax.numpy as jnp
import numpy as np
from jax.experimental import pallas as pl
from jax.experimental.pallas import tpu as pltpu

D_MODEL = 128
MAX_LEN = 512


def _position_indices(shapes):
    D = shapes[0][1] - 1
    idx_list = []
    for s in shapes:
        t = s[1] - 1
        indices = np.concatenate(
            [np.zeros([1]), np.linspace(1, D, t).astype(np.int32)]
        )
        idx_list.append(indices.astype(np.int64))
    return np.concatenate(idx_list, axis=0).astype(np.int32)


_R_BLK = 304


def _make_fused_gather_broadcast(
    total: int, batch: int, vocab: int, ident_steps: int
):
    n_steps = -(-total // _R_BLK)

    def body(idx_ref, emb_ref, out_ref):
        pid = pl.program_id(0)

        def emit(rows):
            out_ref[...] = jnp.broadcast_to(
                rows[:, None, :], (_R_BLK, batch, D_MODEL)
            )

        if ident_steps:
            @pl.when(pid < ident_steps)
            def _ident():
                base = pl.multiple_of(pid * _R_BLK, _R_BLK)
                emit(emb_ref[pl.ds(base, _R_BLK)])

        @pl.when(pid >= ident_steps)
        def _gather():
            cols = jax.lax.broadcasted_iota(jnp.int32, (_R_BLK, vocab), 1)
            onehot = (cols == idx_ref[...]).astype(jnp.float32)
            emit(jnp.dot(
                onehot, emb_ref[...],
                preferred_element_type=jnp.float32,
                precision=jax.lax.Precision.HIGHEST,
            ))

    return pl.pallas_call(
        body,
        grid=(n_steps,),
        in_specs=[
            pl.BlockSpec((_R_BLK, 1), lambda r: (r, 0)),
            pl.BlockSpec((vocab, D_MODEL), lambda r: (0, 0)),
        ],
        out_specs=pl.BlockSpec((_R_BLK, batch, D_MODEL), lambda r: (r, 0, 0)),
        out_shape=jax.ShapeDtypeStruct((total, batch, D_MODEL), jnp.float32),
    )


def _make_manual_pipeline(total: int, batch: int, vocab: int,
                          chunks, n_bufs: int = 6):
    c_max = max(c[1] for c in chunks)

    def body(idx_ref, emb_ref, out_ref, bufs, *sems):
        handles = [None] * n_bufs
        for i, (start, size, ident) in enumerate(chunks):
            k = i % n_bufs
            if handles[k] is not None:
                handles[k].wait()
            if ident:
                rows = emb_ref[pl.ds(start, size)]
            else:
                cols = jax.lax.broadcasted_iota(jnp.int32, (size, vocab), 1)
                onehot = (cols == idx_ref[pl.ds(start, size)]).astype(
                    jnp.float32)
                rows = jnp.dot(
                    onehot, emb_ref[...],
                    preferred_element_type=jnp.float32,
                    precision=jax.lax.Precision.HIGHEST,
                )
            bufs[k, pl.ds(0, size)] = jnp.broadcast_to(
                rows[:, None, :], (size, batch, D_MODEL))
            handles[k] = pltpu.make_async_copy(
                bufs.at[k, pl.ds(0, size)],
                out_ref.at[pl.ds(start, size)],
                sems[k],
            )
            handles[k].start()
        for h in handles:
            if h is not None:
                h.wait()

    return pl.pallas_call(
        body,
        in_specs=[
            pl.BlockSpec(memory_space=pltpu.VMEM),
            pl.BlockSpec(memory_space=pltpu.VMEM),
        ],
        out_specs=pl.BlockSpec(memory_space=pl.ANY),
        out_shape=jax.ShapeDtypeStruct((total, batch, D_MODEL), jnp.float32),
        scratch_shapes=(
            [pltpu.VMEM((n_bufs, c_max, batch, D_MODEL), jnp.float32)]
            + [pltpu.SemaphoreType.DMA] * n_bufs
        ),
    )


def kernel(modal_feat_0, modal_feat_1, modal_feat_2, emb):
    shapes = [modal_feat_0.shape, modal_feat_1.shape, modal_feat_2.shape]
    batch = shapes[0][0]
    idx = _position_indices(shapes)
    total = idx.shape[0]

    idx_col = np.zeros((total, 1), np.int32)
    idx_col[:, 0] = idx

    eq = idx == np.arange(total, dtype=np.int32)
    ident_prefix = total if eq.all() else int(np.argmin(eq))

    chunks = []
    plan = [32, 96, 64, 64, 64, 64, 64, 64, 64, 64, 64, 64, 64, 66]
    pos = 0
    for size in plan:
        size = min(size, total - pos)
        if size <= 0:
            break
        if pos + size > ident_prefix > pos:
            size = ident_prefix - pos
        chunks.append((pos, size, pos + size <= ident_prefix))
        pos += size
    while pos < total:
        size = min(256, total - pos)
        chunks.append((pos, size, pos + size <= ident_prefix))
        pos += size

    fused = _make_manual_pipeline(total, batch, emb.shape[0], chunks)
    out_t = fused(jnp.asarray(idx_col), emb)
    return jnp.transpose(out_t, (1, 0, 2))


# speedup vs baseline: 1.1136x; 1.0015x over previous
import functools

import jax
import jax.numpy as jnp
import numpy as np
from jax.experimental import pallas as pl
from jax.experimental.pallas import tpu as pltpu

D_MODEL = 128
MAX_LEN = 512


def _position_indices(shapes):
    D = shapes[0][1] - 1
    idx_list = []
    for s in shapes:
        t = s[1] - 1
        indices = np.concatenate(
            [np.zeros([1]), np.linspace(1, D, t).astype(np.int32)]
        )
        idx_list.append(indices.astype(np.int64))
    return np.concatenate(idx_list, axis=0).astype(np.int32)


_R_BLK = 304


def _make_fused_gather_broadcast(
    total: int, batch: int, vocab: int, ident_steps: int
):
    n_steps = -(-total // _R_BLK)

    def body(idx_ref, emb_ref, out_ref):
        pid = pl.program_id(0)

        def emit(rows):
            out_ref[...] = jnp.broadcast_to(
                rows[:, None, :], (_R_BLK, batch, D_MODEL)
            )

        if ident_steps:
            @pl.when(pid < ident_steps)
            def _ident():
                base = pl.multiple_of(pid * _R_BLK, _R_BLK)
                emit(emb_ref[pl.ds(base, _R_BLK)])

        @pl.when(pid >= ident_steps)
        def _gather():
            cols = jax.lax.broadcasted_iota(jnp.int32, (_R_BLK, vocab), 1)
            onehot = (cols == idx_ref[...]).astype(jnp.float32)
            emit(jnp.dot(
                onehot, emb_ref[...],
                preferred_element_type=jnp.float32,
                precision=jax.lax.Precision.HIGHEST,
            ))

    return pl.pallas_call(
        body,
        grid=(n_steps,),
        in_specs=[
            pl.BlockSpec((_R_BLK, 1), lambda r: (r, 0)),
            pl.BlockSpec((vocab, D_MODEL), lambda r: (0, 0)),
        ],
        out_specs=pl.BlockSpec((_R_BLK, batch, D_MODEL), lambda r: (r, 0, 0)),
        out_shape=jax.ShapeDtypeStruct((total, batch, D_MODEL), jnp.float32),
    )


def _make_manual_pipeline(total: int, batch: int, vocab: int,
                          chunks, n_bufs: int = 6):
    c_max = max(c[1] for c in chunks)

    def body(idx_ref, emb_ref, out_ref, bufs, *sems):
        handles = [None] * n_bufs
        for i, (start, size, ident) in enumerate(chunks):
            k = i % n_bufs
            if handles[k] is not None:
                handles[k].wait()
            if ident:
                rows = emb_ref[pl.ds(start, size)]
            else:
                cols = jax.lax.broadcasted_iota(jnp.int32, (size, vocab), 1)
                onehot = (cols == idx_ref[pl.ds(start, size)]).astype(
                    jnp.float32)
                rows = jnp.dot(
                    onehot, emb_ref[...],
                    preferred_element_type=jnp.float32,
                    precision=jax.lax.Precision.HIGHEST,
                )
            bufs[k, pl.ds(0, size)] = jnp.broadcast_to(
                rows[:, None, :], (size, batch, D_MODEL))
            handles[k] = pltpu.make_async_copy(
                bufs.at[k, pl.ds(0, size)],
                out_ref.at[pl.ds(start, size)],
                sems[k],
            )
            handles[k].start()
        for h in handles:
            if h is not None:
                h.wait()

    return pl.pallas_call(
        body,
        in_specs=[
            pl.BlockSpec(memory_space=pltpu.VMEM),
            pl.BlockSpec(memory_space=pltpu.VMEM),
        ],
        out_specs=pl.BlockSpec(memory_space=pl.ANY),
        out_shape=jax.ShapeDtypeStruct((total, batch, D_MODEL), jnp.float32),
        scratch_shapes=(
            [pltpu.VMEM((n_bufs, c_max, batch, D_MODEL), jnp.float32)]
            + [pltpu.SemaphoreType.DMA] * n_bufs
        ),
    )


def kernel(modal_feat_0, modal_feat_1, modal_feat_2, emb):
    shapes = [modal_feat_0.shape, modal_feat_1.shape, modal_feat_2.shape]
    batch = shapes[0][0]
    idx = _position_indices(shapes)
    total = idx.shape[0]

    idx_col = np.zeros((total, 1), np.int32)
    idx_col[:, 0] = idx

    eq = idx == np.arange(total, dtype=np.int32)
    ident_prefix = total if eq.all() else int(np.argmin(eq))

    chunks = []
    plan = [32, 32, 64, 96, 96, 96, 96, 98, 96, 96, 96]
    pos = 0
    for size in plan:
        size = min(size, total - pos)
        if size <= 0:
            break
        if pos + size > ident_prefix > pos:
            size = ident_prefix - pos
        chunks.append((pos, size, pos + size <= ident_prefix))
        pos += size
    while pos < total:
        size = min(256, total - pos)
        chunks.append((pos, size, pos + size <= ident_prefix))
        pos += size

    fused = _make_manual_pipeline(total, batch, emb.shape[0], chunks)
    out_t = fused(jnp.asarray(idx_col), emb)
    return jnp.transpose(out_t, (1, 0, 2))


# speedup vs baseline: 1.1237x; 1.0091x over previous
import jax
import jax.numpy as jnp
import numpy as np
from jax.experimental import pallas as pl
from jax.experimental.pallas import tpu as pltpu

D_MODEL = 128
MAX_LEN = 512


def _position_indices(shapes):
    D = shapes[0][1] - 1
    idx_list = []
    for s in shapes:
        t = s[1] - 1
        indices = np.concatenate(
            [np.zeros([1]), np.linspace(1, D, t).astype(np.int32)]
        )
        idx_list.append(indices.astype(np.int64))
    return np.concatenate(idx_list, axis=0).astype(np.int32)


def _make_manual_pipeline(total: int, batch: int, vocab: int,
                          chunks, n_bufs: int = 6):
    c_max = max(c[1] for c in chunks)

    def body(idx_ref, emb_ref, out_ref, bufs, *sems):
        handles = [None] * n_bufs
        for i, (start, size, ident) in enumerate(chunks):
            k = i % n_bufs
            if handles[k] is not None:
                handles[k].wait()
            if ident:
                rows = emb_ref[pl.ds(start, size)]
            else:
                cols = jax.lax.broadcasted_iota(jnp.int32, (size, vocab), 1)
                onehot = (cols == idx_ref[pl.ds(start, size)]).astype(
                    jnp.float32)
                rows = jnp.dot(
                    onehot, emb_ref[...],
                    preferred_element_type=jnp.float32,
                    precision=jax.lax.Precision.HIGHEST,
                )
            bufs[k, pl.ds(0, size)] = jnp.broadcast_to(
                rows[:, None, :], (size, batch, D_MODEL))
            handles[k] = pltpu.make_async_copy(
                bufs.at[k, pl.ds(0, size)],
                out_ref.at[pl.ds(start, size)],
                sems[k],
            )
            handles[k].start()
        for h in handles:
            if h is not None:
                h.wait()

    return pl.pallas_call(
        body,
        in_specs=[
            pl.BlockSpec(memory_space=pltpu.VMEM),
            pl.BlockSpec(memory_space=pltpu.VMEM),
        ],
        out_specs=pl.BlockSpec(memory_space=pl.ANY),
        out_shape=jax.ShapeDtypeStruct((total, batch, D_MODEL), jnp.float32),
        scratch_shapes=(
            [pltpu.VMEM((n_bufs, c_max, batch, D_MODEL), jnp.float32)]
            + [pltpu.SemaphoreType.DMA] * n_bufs
        ),
    )


def kernel(modal_feat_0, modal_feat_1, modal_feat_2, emb):
    shapes = [modal_feat_0.shape, modal_feat_1.shape, modal_feat_2.shape]
    batch = shapes[0][0]
    idx = _position_indices(shapes)
    total = idx.shape[0]

    idx_col = np.zeros((total, 1), np.int32)
    idx_col[:, 0] = idx

    eq = idx == np.arange(total, dtype=np.int32)
    ident_prefix = total if eq.all() else int(np.argmin(eq))

    chunks = []
    plan = [32, 96, 64, 64, 64, 64, 64, 64, 64, 64, 64, 64, 64, 66]
    pos = 0
    for size in plan:
        size = min(size, total - pos)
        if size <= 0:
            break
        if pos + size > ident_prefix > pos:
            size = ident_prefix - pos
        chunks.append((pos, size, pos + size <= ident_prefix))
        pos += size
    while pos < total:
        size = min(256, total - pos)
        chunks.append((pos, size, pos + size <= ident_prefix))
        pos += size

    fused = _make_manual_pipeline(total, batch, emb.shape[0], chunks)
    out_t = fused(jnp.asarray(idx_col), emb)
    return jnp.transpose(out_t, (1, 0, 2))
